# baseline (speedup 1.0000x reference)
import jax
import jax.numpy as jnp
from jax import lax
from jax.experimental import pallas as pl

_SCALE = 1.0 + 0.1
_TILES_PER_BLOCK = 8


def _tc_body(sizes_ref, x_ref, o_ref):
    s = sizes_ref[0, 0, :]
    rows = lax.broadcasted_iota(jnp.int32, (_TILES_PER_BLOCK, 128), 1)
    scale = jnp.where(rows < s[:, None], _SCALE, 0.0).astype(o_ref.dtype)
    o_ref[...] = x_ref[...] * scale[:, :, None]


def kernel(input_tensor, batch_set_size):
    B, T, N, F = input_tensor.shape
    BT = B * T
    x = input_tensor.reshape(BT, N, F)
    sizes = batch_set_size.reshape(BT // _TILES_PER_BLOCK, 1, _TILES_PER_BLOCK)
    grid = (BT // _TILES_PER_BLOCK,)
    out = pl.pallas_call(
        _tc_body,
        grid=grid,
        in_specs=[
            pl.BlockSpec((1, 1, _TILES_PER_BLOCK), lambda i: (i, 0, 0)),
            pl.BlockSpec((_TILES_PER_BLOCK, N, F), lambda i: (i, 0, 0)),
        ],
        out_specs=pl.BlockSpec((_TILES_PER_BLOCK, N, F), lambda i: (i, 0, 0)),
        out_shape=jax.ShapeDtypeStruct((BT, N, F), input_tensor.dtype),
    )(sizes, x)
    return out.reshape(B, T, N, F)
